# baseline (speedup 1.0000x reference)
import functools

import jax
import jax.numpy as jnp
from jax import lax
from jax.experimental import pallas as pl
from jax.experimental.pallas import tpu as pltpu
from jax.experimental.pallas import tpu_sc as plsc

_L = 16
_CHUNK = 128
_NBUF = 4
_ZPAD = 2048
_NC = 2
_NS = 16
_NW = _NC * _NS


def _sc_gather_sum(table, idx4, mask4, nb, bl):
    d = table.shape[1]
    n_half = idx4.shape[1]
    bars_ph = idx4.shape[2]
    n_cols = d // _L

    mesh = plsc.VectorSubcoreMesh(
        core_axis_name="c", subcore_axis_name="s",
        num_cores=_NC, num_subcores=_NS)

    @functools.partial(
        pl.kernel,
        out_type=jax.ShapeDtypeStruct((nb, d), jnp.float32),
        mesh=mesh,
        compiler_params=pltpu.CompilerParams(
            use_tc_tiling_on_sc=False, needs_layout_passes=False),
        scratch_types=[
            pltpu.VMEM((bars_ph, bl), jnp.int32),
            pltpu.VMEM((bars_ph, bl), jnp.float32),
            pltpu.VMEM((_NBUF, bl, d), jnp.bfloat16),
            pltpu.VMEM((bars_ph, d), jnp.float32),
        ] + [pltpu.SemaphoreType.DMA] * _NBUF,
    )
    def body(table_hbm, idx_hbm, mask_hbm, out_hbm,
             idx_v, mask_v, rows_v, out_v, *sems):
        cid = lax.axis_index("c")
        sid = lax.axis_index("s")
        wid = sid * _NC + cid

        for h in range(n_half):
            pltpu.sync_copy(idx_hbm.at[wid, h], idx_v)
            pltpu.sync_copy(mask_hbm.at[wid, h], mask_v)
            for b in range(_NBUF):
                pltpu.async_copy(table_hbm.at[idx_v.at[b]], rows_v.at[b],
                                 sems[b])

            def outer(i, carry):
                for b in range(_NBUF):
                    g = i * _NBUF + b
                    pltpu.make_async_copy(
                        table_hbm.at[idx_v.at[g]], rows_v.at[b],
                        sems[b]).wait()

                    def acc(k, accs, b=b, g=g):
                        res = list(accs)
                        for u in range(_L):
                            r = k * _L + u
                            mrow = plsc.load_gather(
                                mask_v.at[g], [jnp.broadcast_to(r, (_L,))])
                            for j in range(n_cols // 2):
                                ab = rows_v[b, r, pl.ds(j * 2 * _L, 2 * _L)]
                                lo, hi = plsc.unpack(
                                    ab, format=plsc.PackFormat.INTERLEAVED,
                                    preferred_element_type=jnp.float32)
                                res[2 * j] = res[2 * j] + lo * mrow
                                res[2 * j + 1] = res[2 * j + 1] + hi * mrow
                        return tuple(res)

                    a = lax.fori_loop(
                        0, bl // _L, acc,
                        (jnp.zeros((_L,), jnp.float32),) * n_cols)
                    for c in range(n_cols):
                        out_v[g, pl.ds(c * _L, _L)] = a[c]
                    gn = lax.min(g + _NBUF, bars_ph - 1)
                    pltpu.async_copy(table_hbm.at[idx_v.at[gn]],
                                     rows_v.at[b], sems[b])
                return carry

            lax.fori_loop(0, bars_ph // _NBUF, outer, 0)
            for b in range(_NBUF):
                pltpu.make_async_copy(
                    table_hbm.at[idx_v.at[bars_ph - 1]], rows_v.at[b],
                    sems[b]).wait()
            start = wid * (n_half * bars_ph) + h * bars_ph
            pltpu.sync_copy(out_v, out_hbm.at[pl.ds(start, bars_ph)])

    return body(table, idx4, mask4)


def _tc_proj_ln(sums, maskf, wt, bp, g2, b2):
    nb, d = sums.shape
    dm = wt.shape[1]
    rows = 256
    grid = (nb // rows,)

    def body(s_ref, m_ref, w_ref, bp_ref, g_ref, b_ref, o_ref):
        s = s_ref[...]
        m = m_ref[...]
        cnt = jnp.maximum(jnp.sum(m, axis=1, keepdims=True), 1.0)
        mean = s / cnt
        y = jnp.dot(mean, w_ref[...],
                    preferred_element_type=jnp.float32) + bp_ref[...]
        mu = jnp.mean(y, axis=1, keepdims=True)
        yc = y - mu
        var = jnp.mean(yc * yc, axis=1, keepdims=True)
        o_ref[...] = yc * lax.rsqrt(var + 1e-5) * g_ref[...] + b_ref[...]

    return pl.pallas_call(
        body,
        grid=grid,
        in_specs=[
            pl.BlockSpec((rows, d), lambda i: (i, 0)),
            pl.BlockSpec((rows, d), lambda i: (i, 0)),
            pl.BlockSpec((d, dm), lambda i: (0, 0)),
            pl.BlockSpec((rows, dm), lambda i: (0, 0)),
            pl.BlockSpec((1, dm), lambda i: (0, 0)),
            pl.BlockSpec((1, dm), lambda i: (0, 0)),
        ],
        out_specs=pl.BlockSpec((rows, dm), lambda i: (i, 0)),
        out_shape=jax.ShapeDtypeStruct((nb, dm), jnp.float32),
    )(sums, maskf, wt, bp, g2, b2)


def kernel(bar_indices, char_mask, bar_mask, char_table, proj_w, proj_b,
           pos_table, ln_g, ln_b):
    batch, max_bars, bl = bar_indices.shape
    nb = batch * max_bars
    dm = proj_w.shape[0]

    maskf = char_mask.reshape(nb, bl).astype(jnp.float32)
    n_half = 2
    bars_ph = nb // _NW // n_half
    idx4 = bar_indices.reshape(_NW, n_half, bars_ph, bl)
    mask4 = maskf.reshape(_NW, n_half, bars_ph, bl)

    table_bf = char_table.astype(jnp.bfloat16)
    table_flat = jax.lax.optimization_barrier(table_bf.reshape(-1))
    table_lin = table_flat.reshape(char_table.shape)

    sums = _sc_gather_sum(table_lin, idx4, mask4, nb, bl)
    perm = jnp.asarray(
        [j * 32 + p + 2 * u for j in range(bl // 32)
         for p in range(2) for u in range(16)], dtype=jnp.int32)
    wt = proj_w.T[perm, :]
    rows = 256
    bp = jnp.tile(pos_table[:max_bars] + proj_b[None, :],
                  (rows // max_bars, 1))
    out = _tc_proj_ln(sums, maskf, wt, bp,
                      ln_g.reshape(1, dm), ln_b.reshape(1, dm))
    return out.reshape(batch, max_bars, dm), bar_mask


# speedup vs baseline: 1.3915x; 1.3915x over previous
import functools

import jax
import jax.numpy as jnp
from jax import lax
from jax.experimental import pallas as pl
from jax.experimental.pallas import tpu as pltpu
from jax.experimental.pallas import tpu_sc as plsc

_L = 16
_CHUNK = 128
_NBUF = 4
_ZPAD = 2048
_NC = 2
_NS = 16
_NW = _NC * _NS


def _sc_gather_sum(table, idx4, mask4, nb, bl):
    d = table.shape[1]
    n_half = idx4.shape[1]
    bars_ph = idx4.shape[2]
    n_cols = d // _L

    mesh = plsc.VectorSubcoreMesh(
        core_axis_name="c", subcore_axis_name="s",
        num_cores=_NC, num_subcores=_NS)

    @functools.partial(
        pl.kernel,
        out_type=jax.ShapeDtypeStruct((nb, d), jnp.float32),
        mesh=mesh,
        compiler_params=pltpu.CompilerParams(use_tc_tiling_on_sc=False),
        scratch_types=[
            pltpu.VMEM((bars_ph, bl), jnp.int32),
            pltpu.VMEM((bars_ph, bl), jnp.float32),
            pltpu.VMEM((_NBUF, bl, d), jnp.float32),
            pltpu.VMEM((bars_ph, d), jnp.float32),
        ] + [pltpu.SemaphoreType.DMA] * _NBUF,
    )
    def body(table_hbm, idx_hbm, mask_hbm, out_hbm,
             idx_v, mask_v, rows_v, out_v, *sems):
        cid = lax.axis_index("c")
        sid = lax.axis_index("s")
        wid = sid * _NC + cid

        for h in range(n_half):
            pltpu.sync_copy(idx_hbm.at[wid, h], idx_v)
            pltpu.sync_copy(mask_hbm.at[wid, h], mask_v)
            for b in range(_NBUF):
                pltpu.async_copy(table_hbm.at[idx_v.at[b]], rows_v.at[b],
                                 sems[b])

            def outer(i, carry):
                for b in range(_NBUF):
                    g = i * _NBUF + b
                    pltpu.make_async_copy(
                        table_hbm.at[idx_v.at[g]], rows_v.at[b],
                        sems[b]).wait()

                    def acc(k, accs, b=b, g=g):
                        res = list(accs)
                        mvec = mask_v[g, pl.ds(k * _L, _L)]
                        for u in range(_L):
                            r = k * _L + u
                            mrow = jnp.broadcast_to(mvec[u:u + 1], (_L,))
                            for c in range(n_cols):
                                res[c] = res[c] + rows_v[
                                    b, r, pl.ds(c * _L, _L)] * mrow
                        return tuple(res)

                    a = lax.fori_loop(
                        0, bl // _L, acc,
                        (jnp.zeros((_L,), jnp.float32),) * n_cols)
                    for c in range(n_cols):
                        out_v[g, pl.ds(c * _L, _L)] = a[c]
                    gn = lax.min(g + _NBUF, bars_ph - 1)
                    pltpu.async_copy(table_hbm.at[idx_v.at[gn]],
                                     rows_v.at[b], sems[b])
                return carry

            lax.fori_loop(0, bars_ph // _NBUF, outer, 0)
            for b in range(_NBUF):
                pltpu.make_async_copy(
                    table_hbm.at[idx_v.at[bars_ph - 1]], rows_v.at[b],
                    sems[b]).wait()
            start = wid * (n_half * bars_ph) + h * bars_ph
            pltpu.sync_copy(out_v, out_hbm.at[pl.ds(start, bars_ph)])

    return body(table, idx4, mask4)


def _tc_proj_ln(sums, maskf, wt, bp, g2, b2):
    nb, d = sums.shape
    dm = wt.shape[1]
    rows = 256
    grid = (nb // rows,)

    def body(s_ref, m_ref, w_ref, bp_ref, g_ref, b_ref, o_ref):
        s = s_ref[...]
        m = m_ref[...]
        cnt = jnp.maximum(jnp.sum(m, axis=1, keepdims=True), 1.0)
        mean = s / cnt
        y = jnp.dot(mean, w_ref[...],
                    preferred_element_type=jnp.float32) + bp_ref[...]
        mu = jnp.mean(y, axis=1, keepdims=True)
        yc = y - mu
        var = jnp.mean(yc * yc, axis=1, keepdims=True)
        o_ref[...] = yc * lax.rsqrt(var + 1e-5) * g_ref[...] + b_ref[...]

    return pl.pallas_call(
        body,
        grid=grid,
        in_specs=[
            pl.BlockSpec((rows, d), lambda i: (i, 0)),
            pl.BlockSpec((rows, d), lambda i: (i, 0)),
            pl.BlockSpec((d, dm), lambda i: (0, 0)),
            pl.BlockSpec((rows, dm), lambda i: (0, 0)),
            pl.BlockSpec((1, dm), lambda i: (0, 0)),
            pl.BlockSpec((1, dm), lambda i: (0, 0)),
        ],
        out_specs=pl.BlockSpec((rows, dm), lambda i: (i, 0)),
        out_shape=jax.ShapeDtypeStruct((nb, dm), jnp.float32),
    )(sums, maskf, wt, bp, g2, b2)


def kernel(bar_indices, char_mask, bar_mask, char_table, proj_w, proj_b,
           pos_table, ln_g, ln_b):
    batch, max_bars, bl = bar_indices.shape
    nb = batch * max_bars
    dm = proj_w.shape[0]

    maskf = char_mask.reshape(nb, bl).astype(jnp.float32)
    n_half = 2
    bars_ph = nb // _NW // n_half
    idx4 = bar_indices.reshape(_NW, n_half, bars_ph, bl)
    mask4 = maskf.reshape(_NW, n_half, bars_ph, bl)

    table_flat = jax.lax.optimization_barrier(char_table.reshape(-1))
    table_lin = table_flat.reshape(char_table.shape)

    sums = _sc_gather_sum(table_lin, idx4, mask4, nb, bl)
    wt = proj_w.T
    rows = 256
    bp = jnp.tile(pos_table[:max_bars] + proj_b[None, :],
                  (rows // max_bars, 1))
    out = _tc_proj_ln(sums, maskf, wt, bp,
                      ln_g.reshape(1, dm), ln_b.reshape(1, dm))
    return out.reshape(batch, max_bars, dm), bar_mask


# speedup vs baseline: 1.5045x; 1.0812x over previous
import functools

import jax
import jax.numpy as jnp
from jax import lax
from jax.experimental import pallas as pl
from jax.experimental.pallas import tpu as pltpu
from jax.experimental.pallas import tpu_sc as plsc

_L = 16
_CHUNK = 128
_NBUF = 8
_ZPAD = 2048
_NC = 2
_NS = 16
_NW = _NC * _NS


def _sc_gather_sum(table, idx4, mask4, nb, bl):
    d = table.shape[1]
    n_half = idx4.shape[1]
    bars_ph = idx4.shape[2]
    n_cols = d // _L

    mesh = plsc.VectorSubcoreMesh(
        core_axis_name="c", subcore_axis_name="s",
        num_cores=_NC, num_subcores=_NS)

    @functools.partial(
        pl.kernel,
        out_type=jax.ShapeDtypeStruct((nb, d), jnp.float32),
        mesh=mesh,
        compiler_params=pltpu.CompilerParams(use_tc_tiling_on_sc=False),
        scratch_types=[
            pltpu.VMEM((bars_ph, bl), jnp.int32),
            pltpu.VMEM((bars_ph, bl), jnp.float32),
            pltpu.VMEM((_NBUF, bl, d), jnp.float32),
            pltpu.VMEM((bars_ph, d), jnp.float32),
        ] + [pltpu.SemaphoreType.DMA] * _NBUF,
    )
    def body(table_hbm, idx_hbm, mask_hbm, out_hbm,
             idx_v, mask_v, rows_v, out_v, *sems):
        cid = lax.axis_index("c")
        sid = lax.axis_index("s")
        wid = sid * _NC + cid

        for h in range(n_half):
            pltpu.sync_copy(idx_hbm.at[wid, h], idx_v)
            pltpu.sync_copy(mask_hbm.at[wid, h], mask_v)
            for b in range(_NBUF):
                pltpu.async_copy(table_hbm.at[idx_v.at[b]], rows_v.at[b],
                                 sems[b])

            def outer(i, carry):
                for b in range(_NBUF):
                    g = i * _NBUF + b
                    pltpu.make_async_copy(
                        table_hbm.at[idx_v.at[g]], rows_v.at[b],
                        sems[b]).wait()

                    def acc(k, accs, b=b, g=g):
                        res = list(accs)
                        mvec = mask_v[g, pl.ds(k * _L, _L)]
                        for u in range(_L):
                            r = k * _L + u
                            mrow = jnp.broadcast_to(mvec[u:u + 1], (_L,))
                            for c in range(n_cols):
                                res[c] = res[c] + rows_v[
                                    b, r, pl.ds(c * _L, _L)] * mrow
                        return tuple(res)

                    a = lax.fori_loop(
                        0, bl // _L, acc,
                        (jnp.zeros((_L,), jnp.float32),) * n_cols)
                    for c in range(n_cols):
                        out_v[g, pl.ds(c * _L, _L)] = a[c]
                    gn = lax.min(g + _NBUF, bars_ph - 1)
                    pltpu.async_copy(table_hbm.at[idx_v.at[gn]],
                                     rows_v.at[b], sems[b])
                return carry

            lax.fori_loop(0, bars_ph // _NBUF, outer, 0)
            for b in range(_NBUF):
                pltpu.make_async_copy(
                    table_hbm.at[idx_v.at[bars_ph - 1]], rows_v.at[b],
                    sems[b]).wait()
            start = wid * (n_half * bars_ph) + h * bars_ph
            pltpu.sync_copy(out_v, out_hbm.at[pl.ds(start, bars_ph)])

    return body(table, idx4, mask4)


def _tc_proj_ln(sums, maskf, wt, bp, g2, b2):
    nb, d = sums.shape
    dm = wt.shape[1]
    rows = 256
    grid = (nb // rows,)

    def body(s_ref, m_ref, w_ref, bp_ref, g_ref, b_ref, o_ref):
        s = s_ref[...]
        m = m_ref[...]
        cnt = jnp.maximum(jnp.sum(m, axis=1, keepdims=True), 1.0)
        mean = s / cnt
        y = jnp.dot(mean, w_ref[...],
                    preferred_element_type=jnp.float32) + bp_ref[...]
        mu = jnp.mean(y, axis=1, keepdims=True)
        yc = y - mu
        var = jnp.mean(yc * yc, axis=1, keepdims=True)
        o_ref[...] = yc * lax.rsqrt(var + 1e-5) * g_ref[...] + b_ref[...]

    return pl.pallas_call(
        body,
        grid=grid,
        in_specs=[
            pl.BlockSpec((rows, d), lambda i: (i, 0)),
            pl.BlockSpec((rows, d), lambda i: (i, 0)),
            pl.BlockSpec((d, dm), lambda i: (0, 0)),
            pl.BlockSpec((rows, dm), lambda i: (0, 0)),
            pl.BlockSpec((1, dm), lambda i: (0, 0)),
            pl.BlockSpec((1, dm), lambda i: (0, 0)),
        ],
        out_specs=pl.BlockSpec((rows, dm), lambda i: (i, 0)),
        out_shape=jax.ShapeDtypeStruct((nb, dm), jnp.float32),
    )(sums, maskf, wt, bp, g2, b2)


def kernel(bar_indices, char_mask, bar_mask, char_table, proj_w, proj_b,
           pos_table, ln_g, ln_b):
    batch, max_bars, bl = bar_indices.shape
    nb = batch * max_bars
    dm = proj_w.shape[0]

    maskf = char_mask.reshape(nb, bl).astype(jnp.float32)
    n_half = 4
    bars_ph = nb // _NW // n_half
    idx4 = bar_indices.reshape(_NW, n_half, bars_ph, bl)
    mask4 = maskf.reshape(_NW, n_half, bars_ph, bl)

    table_flat = jax.lax.optimization_barrier(char_table.reshape(-1))
    table_lin = table_flat.reshape(char_table.shape)

    sums = _sc_gather_sum(table_lin, idx4, mask4, nb, bl)
    wt = proj_w.T
    rows = 256
    bp = jnp.tile(pos_table[:max_bars] + proj_b[None, :],
                  (rows // max_bars, 1))
    out = _tc_proj_ln(sums, maskf, wt, bp,
                      ln_g.reshape(1, dm), ln_b.reshape(1, dm))
    return out.reshape(batch, max_bars, dm), bar_mask


# speedup vs baseline: 1.5057x; 1.0008x over previous
import functools

import jax
import jax.numpy as jnp
from jax import lax
from jax.experimental import pallas as pl
from jax.experimental.pallas import tpu as pltpu
from jax.experimental.pallas import tpu_sc as plsc

_L = 16
_NBUF = 8
_NC = 2
_NS = 16
_NW = _NC * _NS


def _sc_gather_sum(table, idx4, mask4, nb, bl):
    d = table.shape[1]
    n_half = idx4.shape[1]
    bars_ph = idx4.shape[2]
    n_cols = d // _L

    mesh = plsc.VectorSubcoreMesh(
        core_axis_name="c", subcore_axis_name="s",
        num_cores=_NC, num_subcores=_NS)

    @functools.partial(
        pl.kernel,
        out_type=jax.ShapeDtypeStruct((nb, d), jnp.float32),
        mesh=mesh,
        compiler_params=pltpu.CompilerParams(use_tc_tiling_on_sc=False),
        scratch_types=[
            pltpu.VMEM((bars_ph, bl), jnp.int32),
            pltpu.VMEM((bars_ph, bl), jnp.float32),
            pltpu.VMEM((_NBUF, bl, d), jnp.float32),
            pltpu.VMEM((bars_ph, d), jnp.float32),
        ] + [pltpu.SemaphoreType.DMA] * _NBUF,
    )
    def body(table_hbm, idx_hbm, mask_hbm, out_hbm,
             idx_v, mask_v, rows_v, out_v, *sems):
        cid = lax.axis_index("c")
        sid = lax.axis_index("s")
        wid = sid * _NC + cid

        for h in range(n_half):
            pltpu.sync_copy(idx_hbm.at[wid, h], idx_v)
            pltpu.sync_copy(mask_hbm.at[wid, h], mask_v)
            for b in range(_NBUF):
                pltpu.async_copy(table_hbm.at[idx_v.at[b]], rows_v.at[b],
                                 sems[b])

            def outer(i, carry):
                for b in range(_NBUF):
                    g = i * _NBUF + b
                    pltpu.make_async_copy(
                        table_hbm.at[idx_v.at[g]], rows_v.at[b],
                        sems[b]).wait()

                    def acc(k, accs, b=b, g=g):
                        res = list(accs)
                        mvec = mask_v[g, pl.ds(k * _L, _L)]
                        for u in range(_L):
                            r = k * _L + u
                            mrow = jnp.broadcast_to(mvec[u:u + 1], (_L,))
                            for c in range(n_cols):
                                res[c] = res[c] + rows_v[
                                    b, r, pl.ds(c * _L, _L)] * mrow
                        return tuple(res)

                    a = lax.fori_loop(
                        0, bl // _L, acc,
                        (jnp.zeros((_L,), jnp.float32),) * n_cols)
                    for c in range(n_cols):
                        out_v[g, pl.ds(c * _L, _L)] = a[c]
                    gn = lax.min(g + _NBUF, bars_ph - 1)
                    pltpu.async_copy(table_hbm.at[idx_v.at[gn]],
                                     rows_v.at[b], sems[b])
                return carry

            lax.fori_loop(0, bars_ph // _NBUF, outer, 0)
            for b in range(_NBUF):
                pltpu.make_async_copy(
                    table_hbm.at[idx_v.at[bars_ph - 1]], rows_v.at[b],
                    sems[b]).wait()
            start = wid * (n_half * bars_ph) + h * bars_ph
            pltpu.sync_copy(out_v, out_hbm.at[pl.ds(start, bars_ph)])

    return body(table, idx4, mask4)


def _tc_proj_ln(sums, maskf, wt, bp, g2, b2):
    nb, d = sums.shape
    dm = wt.shape[1]
    rows = 256
    grid = (nb // rows,)

    def body(s_ref, m_ref, w_ref, bp_ref, g_ref, b_ref, o_ref):
        s = s_ref[...]
        m = m_ref[...]
        cnt = jnp.maximum(jnp.sum(m, axis=1, keepdims=True), 1.0)
        mean = s / cnt
        y = jnp.dot(mean, w_ref[...],
                    preferred_element_type=jnp.float32) + bp_ref[...]
        mu = jnp.mean(y, axis=1, keepdims=True)
        yc = y - mu
        var = jnp.mean(yc * yc, axis=1, keepdims=True)
        o_ref[...] = yc * lax.rsqrt(var + 1e-5) * g_ref[...] + b_ref[...]

    return pl.pallas_call(
        body,
        grid=grid,
        in_specs=[
            pl.BlockSpec((rows, d), lambda i: (i, 0)),
            pl.BlockSpec((rows, d), lambda i: (i, 0)),
            pl.BlockSpec((d, dm), lambda i: (0, 0)),
            pl.BlockSpec((rows, dm), lambda i: (0, 0)),
            pl.BlockSpec((1, dm), lambda i: (0, 0)),
            pl.BlockSpec((1, dm), lambda i: (0, 0)),
        ],
        out_specs=pl.BlockSpec((rows, dm), lambda i: (i, 0)),
        out_shape=jax.ShapeDtypeStruct((nb, dm), jnp.float32),
    )(sums, maskf, wt, bp, g2, b2)


def kernel(bar_indices, char_mask, bar_mask, char_table, proj_w, proj_b,
           pos_table, ln_g, ln_b):
    batch, max_bars, bl = bar_indices.shape
    nb = batch * max_bars
    dm = proj_w.shape[0]

    maskf = char_mask.reshape(nb, bl).astype(jnp.float32)
    n_half = 4
    bars_ph = nb // _NW // n_half
    idx4 = bar_indices.reshape(_NW, n_half, bars_ph, bl)
    mask4 = maskf.reshape(_NW, n_half, bars_ph, bl)

    table_flat = jax.lax.optimization_barrier(char_table.reshape(-1))
    table_lin = table_flat.reshape(char_table.shape)

    sums = _sc_gather_sum(table_lin, idx4, mask4, nb, bl)
    wt = proj_w.T
    rows = 256
    bp = jnp.tile(pos_table[:max_bars] + proj_b[None, :],
                  (rows // max_bars, 1))
    out = _tc_proj_ln(sums, maskf, wt, bp,
                      ln_g.reshape(1, dm), ln_b.reshape(1, dm))
    return out.reshape(batch, max_bars, dm), bar_mask
